# baseline (speedup 1.0000x reference)
import functools

import jax
import jax.numpy as jnp
from jax import lax
from jax.experimental import pallas as pl
from jax.experimental.pallas import tpu as pltpu
from jax.experimental.pallas import tpu_sc as plsc

D = 64
NW = 32
CHUNK = 128
STREAMS_PER_BUF = 4
BUF_ROWS = CHUNK * STREAMS_PER_BUF


def _emb_call(total):
    b_per_w = total // NW
    n_rows = b_per_w // CHUNK
    n_bufs = b_per_w // BUF_ROWS

    mesh = plsc.VectorSubcoreMesh(core_axis_name="c", subcore_axis_name="s")

    @functools.partial(
        pl.kernel,
        mesh=mesh,
        out_type=jax.ShapeDtypeStruct((total, D), jnp.float32),
        compiler_params=pltpu.CompilerParams(use_tc_tiling_on_sc=False),
        scratch_types=[
            pltpu.VMEM((n_rows, CHUNK), jnp.int32),
            pltpu.VMEM((BUF_ROWS, D), jnp.float32),
            pltpu.SemaphoreType.DMA,
        ],
    )
    def emb(idx_hbm, table_hbm, out_hbm, idx_v, rows_v, gsem):
        wid = lax.axis_index("s") * 2 + lax.axis_index("c")
        base = wid * b_per_w
        pltpu.sync_copy(idx_hbm.at[wid], idx_v)

        def body(g, carry):
            copies = []
            for j in range(STREAMS_PER_BUF):
                row = g * STREAMS_PER_BUF + j
                cp = pltpu.make_async_copy(
                    table_hbm.at[idx_v.at[row]],
                    rows_v.at[pl.ds(j * CHUNK, CHUNK)],
                    gsem,
                )
                cp.start()
                copies.append(cp)
            for cp in copies:
                cp.wait()
            pltpu.sync_copy(rows_v, out_hbm.at[pl.ds(base + g * BUF_ROWS, BUF_ROWS)])
            return carry

        lax.fori_loop(0, n_bufs, body, 0)

    return emb


def kernel(token_ids, weight):
    B, S = token_ids.shape
    total = B * S
    idx = token_ids.reshape(NW, total // (NW * CHUNK), CHUNK).astype(jnp.int32)
    out = _emb_call(total)(idx, weight)
    return out.reshape(B, S, D)


# speedup vs baseline: 1.0205x; 1.0205x over previous
import functools

import jax
import jax.numpy as jnp
from jax import lax
from jax.experimental import pallas as pl
from jax.experimental.pallas import tpu as pltpu
from jax.experimental.pallas import tpu_sc as plsc

D = 64
NW = 32
CHUNK = 128
STREAMS_PER_BUF = 4
BUF_ROWS = CHUNK * STREAMS_PER_BUF


def _emb_call(total):
    b_per_w = total // NW
    n_rows = b_per_w // CHUNK
    n_bufs = b_per_w // BUF_ROWS

    mesh = plsc.VectorSubcoreMesh(core_axis_name="c", subcore_axis_name="s")

    @functools.partial(
        pl.kernel,
        mesh=mesh,
        out_type=jax.ShapeDtypeStruct((total, D), jnp.float32),
        compiler_params=pltpu.CompilerParams(use_tc_tiling_on_sc=False),
        scratch_types=[
            pltpu.VMEM((n_rows, CHUNK), jnp.int32),
            pltpu.VMEM((BUF_ROWS, D), jnp.float32),
            pltpu.VMEM((BUF_ROWS, D), jnp.float32),
            pltpu.SemaphoreType.DMA,
            pltpu.SemaphoreType.DMA,
        ],
    )
    def emb(idx_hbm, table_hbm, out_hbm, idx_v, rows0, rows1, g0, g1):
        wid = lax.axis_index("s") * 2 + lax.axis_index("c")
        base = wid * b_per_w
        pltpu.sync_copy(idx_hbm.at[wid], idx_v)

        rows = (rows0, rows1)
        gsem = (g0, g1)

        def fire(g, rows_ref, sem):
            for j in range(STREAMS_PER_BUF):
                pltpu.make_async_copy(
                    table_hbm.at[idx_v.at[g * STREAMS_PER_BUF + j]],
                    rows_ref.at[pl.ds(j * CHUNK, CHUNK)],
                    sem,
                ).start()

        def drain(rows_ref, sem):
            pltpu.make_async_copy(
                table_hbm.at[pl.ds(0, BUF_ROWS)], rows_ref, sem
            ).wait()

        fire(0, rows0, g0)

        def body(p, carry):
            for b in range(2):
                g = p * 2 + b
                drain(rows[b], gsem[b])
                if b == 0:
                    fire(g + 1, rows[1], gsem[1])
                else:
                    @pl.when(g + 1 < n_bufs)
                    def _():
                        fire(g + 1, rows[0], gsem[0])
                pltpu.sync_copy(
                    rows[b], out_hbm.at[pl.ds(base + g * BUF_ROWS, BUF_ROWS)]
                )
            return carry

        lax.fori_loop(0, n_bufs // 2, body, 0)

    return emb


def kernel(token_ids, weight):
    B, S = token_ids.shape
    total = B * S
    idx = token_ids.reshape(NW, total // (NW * CHUNK), CHUNK).astype(jnp.int32)
    out = _emb_call(total)(idx, weight)
    return out.reshape(B, S, D)


# speedup vs baseline: 1.3613x; 1.3339x over previous
import functools

import jax
import jax.numpy as jnp
from jax import lax
from jax.experimental import pallas as pl
from jax.experimental.pallas import tpu as pltpu
from jax.experimental.pallas import tpu_sc as plsc

D = 64
NW = 32
CHUNK = 128
STREAMS_PER_BUF = 4
BUF_ROWS = CHUNK * STREAMS_PER_BUF


def _emb_call(total):
    b_per_w = total // NW
    n_rows = b_per_w // CHUNK
    n_bufs = b_per_w // BUF_ROWS

    mesh = plsc.VectorSubcoreMesh(core_axis_name="c", subcore_axis_name="s")

    @functools.partial(
        pl.kernel,
        mesh=mesh,
        out_type=jax.ShapeDtypeStruct((total, 2 * D), jnp.float32),
        compiler_params=pltpu.CompilerParams(use_tc_tiling_on_sc=False),
        scratch_types=[
            pltpu.VMEM((n_rows, CHUNK), jnp.int32),
            pltpu.VMEM((BUF_ROWS, D), jnp.float32),
            pltpu.VMEM((BUF_ROWS, D), jnp.float32),
            pltpu.SemaphoreType.DMA,
            pltpu.SemaphoreType.DMA,
        ],
    )
    def emb(idx_hbm, table_hbm, out_hbm, idx_v, rows0, rows1, g0, g1):
        wid = lax.axis_index("s") * 2 + lax.axis_index("c")
        base = wid * b_per_w
        pltpu.sync_copy(idx_hbm.at[wid], idx_v)

        rows = (rows0, rows1)
        gsem = (g0, g1)

        def fire(g, rows_ref, sem):
            for j in range(STREAMS_PER_BUF):
                pltpu.make_async_copy(
                    table_hbm.at[idx_v.at[g * STREAMS_PER_BUF + j]],
                    rows_ref.at[pl.ds(j * CHUNK, CHUNK)],
                    sem,
                ).start()

        def drain(rows_ref, sem):
            pltpu.make_async_copy(
                table_hbm.at[pl.ds(0, BUF_ROWS)], rows_ref, sem
            ).wait()

        fire(0, rows0, g0)

        def body(p, carry):
            for b in range(2):
                g = p * 2 + b
                drain(rows[b], gsem[b])
                if b == 0:
                    fire(g + 1, rows[1], gsem[1])
                else:
                    @pl.when(g + 1 < n_bufs)
                    def _():
                        fire(g + 1, rows[0], gsem[0])
                pltpu.sync_copy(
                    rows[b],
                    out_hbm.at[pl.ds(base + g * BUF_ROWS, BUF_ROWS), pl.ds(0, D)],
                )
            return carry

        lax.fori_loop(0, n_bufs // 2, body, 0)

    return emb


def kernel(token_ids, weight):
    B, S = token_ids.shape
    total = B * S
    idx = token_ids.reshape(NW, total // (NW * CHUNK), CHUNK).astype(jnp.int32)
    out = _emb_call(total)(idx, weight)
    return out[:, :D].reshape(B, S, D)
